# baseline (speedup 1.0000x reference)
import functools
import jax
import jax.numpy as jnp
from jax import lax
from jax.experimental import pallas as pl
from jax.experimental.pallas import tpu as pltpu
from jax.experimental.pallas import tpu_sc as plsc

_C = 32
_U = 8
_NW = 32


def _make_sc_call(b, s, d):
    rows_w = s // _NW
    nchunks = rows_w // _C
    n_items = nchunks * b
    nvec = d // 16
    mesh = plsc.VectorSubcoreMesh(core_axis_name="c", subcore_axis_name="s")

    @functools.partial(
        pl.kernel,
        mesh=mesh,
        out_type=jax.ShapeDtypeStruct((b, s, d), jnp.float32),
        scratch_types=[
            pltpu.VMEM((_C, d), jnp.float32),
            pltpu.VMEM((_C, d), jnp.float32),
            pltpu.VMEM((_C, d), jnp.float32),
            pltpu.SemaphoreType.DMA,
            pltpu.SemaphoreType.DMA,
            pltpu.SemaphoreType.DMA,
            pltpu.SemaphoreType.DMA,
        ],
    )
    def k(in_hbm, emb_hbm, out_hbm, emb_v, io0, io1, isem0, isem1, osem0, osem1):
        wid = lax.axis_index("s") * 2 + lax.axis_index("c")
        base = wid * rows_w
        io = (io0, io1)
        isem = (isem0, isem1)
        osem = (osem0, osem1)

        def add_chunk(buf):
            def row_body(r, _):
                for u in range(nvec):
                    sl = pl.ds(u * 16, 16)
                    buf[r, sl] = buf[r, sl] + emb_v[r, sl]
                return _
            lax.fori_loop(0, _C, row_body, 0)

        def item_src(t):
            i, bb = divmod(t, b)
            return (bb, base + i * _C)

        bb0, r0 = item_src(0)
        in_h = [None, None]
        out_h = [None, None]
        in_h[0] = pltpu.async_copy(in_hbm.at[bb0, pl.ds(r0, _C)], io0, isem0)

        for t in range(n_items):
            p = t % 2
            i, bb = divmod(t, b)
            if bb == 0:
                pltpu.sync_copy(emb_hbm.at[pl.ds(base + i * _C, _C)], emb_v)
            in_h[p].wait()
            add_chunk(io[p])
            out_h[p] = pltpu.async_copy(
                io[p], out_hbm.at[bb, pl.ds(base + i * _C, _C)], osem[p]
            )
            if t + 1 < n_items:
                q = (t + 1) % 2
                if out_h[q] is not None:
                    out_h[q].wait()
                bb1, r1 = item_src(t + 1)
                in_h[q] = pltpu.async_copy(
                    in_hbm.at[bb1, pl.ds(r1, _C)], io[q], isem[q]
                )

        out_h[(n_items - 1) % 2].wait()
        if out_h[n_items % 2] is not None:
            out_h[n_items % 2].wait()

    return k


def kernel(inputs, embeddings):
    b, s, d = inputs.shape
    return _make_sc_call(b, s, d)(inputs, embeddings[:s])


# speedup vs baseline: 1.7000x; 1.7000x over previous
import functools
import jax
import jax.numpy as jnp
from jax import lax
from jax.experimental import pallas as pl
from jax.experimental.pallas import tpu as pltpu
from jax.experimental.pallas import tpu_sc as plsc

_C = 8
_NS = 3
_U = 8
_NW = 32


def _make_sc_call(b, s, d):
    rows_w = s // _NW
    nchunks = rows_w // _C
    nvec = d // 16
    npu = nvec // _U
    mesh = plsc.VectorSubcoreMesh(core_axis_name="c", subcore_axis_name="s")

    @functools.partial(
        pl.kernel,
        mesh=mesh,
        out_type=jax.ShapeDtypeStruct((b, s, d), jnp.float32),
        scratch_types=[
            pltpu.VMEM((_NS, b, _C, d), jnp.float32),
            pltpu.VMEM((_NS, _C, d), jnp.float32),
            pltpu.SemaphoreType.DMA,
            pltpu.SemaphoreType.DMA,
            pltpu.SemaphoreType.DMA,
            pltpu.SemaphoreType.DMA,
            pltpu.SemaphoreType.DMA,
            pltpu.SemaphoreType.DMA,
        ],
    )
    def k(in_hbm, emb_hbm, out_hbm, io, emb_v, *sems):
        isem = sems[:_NS]
        osem = sems[_NS:2 * _NS]
        wid = lax.axis_index("s") * 2 + lax.axis_index("c")
        base = wid * rows_w

        def stage_chunk(i, p):
            r0 = base + i * _C
            hs = [pltpu.async_copy(emb_hbm.at[pl.ds(r0, _C)], emb_v.at[p],
                                   isem[p])]
            for bb in range(b):
                hs.append(
                    pltpu.async_copy(in_hbm.at[bb, pl.ds(r0, _C)],
                                     io.at[p, bb], isem[p])
                )
            return hs

        def add_chunk(p):
            def body(t, _):
                r = t // npu
                c0 = (t % npu) * (16 * _U)
                for u in range(_U):
                    sl = pl.ds(c0 + u * 16, 16)
                    e = emb_v[p, r, sl]
                    for bb in range(b):
                        io[p, bb, r, sl] = io[p, bb, r, sl] + e
                return _
            lax.fori_loop(0, _C * npu, body, 0)

        def drain_chunk(i, p):
            r0 = base + i * _C
            return [
                pltpu.async_copy(io.at[p, bb], out_hbm.at[bb, pl.ds(r0, _C)],
                                 osem[p])
                for bb in range(b)
            ]

        in_h = [None] * _NS
        out_h = [None] * _NS
        for i in range(min(_NS - 1, nchunks)):
            in_h[i % _NS] = stage_chunk(i, i % _NS)

        for i in range(nchunks):
            p = i % _NS
            for h in in_h[p]:
                h.wait()
            add_chunk(p)
            out_h[p] = drain_chunk(i, p)
            if i + _NS - 1 < nchunks:
                q = (i + _NS - 1) % _NS
                if out_h[q] is not None:
                    for h in out_h[q]:
                        h.wait()
                in_h[q] = stage_chunk(i + _NS - 1, q)

        for hs in out_h:
            if hs is not None:
                for h in hs:
                    h.wait()

    return k


def kernel(inputs, embeddings):
    b, s, d = inputs.shape
    return _make_sc_call(b, s, d)(inputs, embeddings[:s])


# speedup vs baseline: 2.4878x; 1.4635x over previous
import jax
import jax.numpy as jnp
from jax.experimental import pallas as pl

_SEQ_BLK = 512


def _add_kernel(x_ref, e_ref, o_ref):
    o_ref[...] = x_ref[...] + e_ref[...]


def kernel(inputs, embeddings):
    b, s, d = inputs.shape
    emb = embeddings[:s]
    num_seq = s // _SEQ_BLK
    return pl.pallas_call(
        _add_kernel,
        grid=(num_seq,),
        in_specs=[
            pl.BlockSpec((b, _SEQ_BLK, d), lambda i: (0, i, 0)),
            pl.BlockSpec((_SEQ_BLK, d), lambda i: (i, 0)),
        ],
        out_specs=pl.BlockSpec((b, _SEQ_BLK, d), lambda i: (0, i, 0)),
        out_shape=jax.ShapeDtypeStruct((b, s, d), inputs.dtype),
    )(inputs, emb)
